# baseline (speedup 1.0000x reference)
import jax
import jax.numpy as jnp
from jax.experimental import pallas as pl
from jax.experimental.pallas import tpu as pltpu

N_KV_HEADS = 8
HEAD_DIM = 128
MAX_SEQ_LEN = 8192
Q_LEN = 16
NROWS = N_KV_HEADS * Q_LEN

NBUF = 8
K = 4
TOT = 2 * N_KV_HEADS


def _update_body(pos_ref, kc_ref, vc_ref, kval_ref, vval_ref, ko_ref, vo_ref,
                 bufs, kvv, vvv, in_sems, out_sems, vsem):
    cpk = pltpu.make_async_copy(kval_ref, kvv, vsem)
    cpv = pltpu.make_async_copy(vval_ref, vvv, vsem)
    cpk.start()
    cpv.start()
    cpk.wait()
    cpv.wait()

    def mk_in(c):
        slot = c % NBUF
        src = kc_ref.at[c] if c < N_KV_HEADS else vc_ref.at[c - N_KV_HEADS]
        return pltpu.make_async_copy(src, bufs.at[slot], in_sems.at[slot])

    def mk_out(c):
        slot = c % NBUF
        dst = ko_ref.at[c] if c < N_KV_HEADS else vo_ref.at[c - N_KV_HEADS]
        return pltpu.make_async_copy(bufs.at[slot], dst, out_sems.at[slot])

    in_cp, out_cp = {}, {}
    for c in range(K):
        in_cp[c] = mk_in(c)
        in_cp[c].start()

    for c in range(TOT):
        slot = c % NBUF
        in_cp[c].wait()
        vals = kvv if c < N_KV_HEADS else vvv
        h = c % N_KV_HEADS
        for i in range(Q_LEN):
            p = pos_ref[i]
            bufs[slot, pl.ds(p, 1), :] = vals[pl.ds(h * Q_LEN + i, 1), :]
        out_cp[c] = mk_out(c)
        out_cp[c].start()
        n = c + K
        if n < TOT:
            if n >= NBUF:
                out_cp[n - NBUF].wait()
            in_cp[n] = mk_in(n)
            in_cp[n].start()

    for c in range(TOT - NBUF, TOT):
        out_cp[c].wait()


def kernel(k_cache, v_cache, input_pos, k_val, v_val):
    kc = k_cache.reshape(N_KV_HEADS, MAX_SEQ_LEN, HEAD_DIM)
    vc = v_cache.reshape(N_KV_HEADS, MAX_SEQ_LEN, HEAD_DIM)
    kv = k_val.reshape(NROWS, HEAD_DIM)
    vv = v_val.reshape(NROWS, HEAD_DIM)
    pos = input_pos.astype(jnp.int32)

    any_spec = pl.BlockSpec(memory_space=pl.ANY)

    grid_spec = pltpu.PrefetchScalarGridSpec(
        num_scalar_prefetch=1,
        grid=(1,),
        in_specs=[any_spec, any_spec, any_spec, any_spec],
        out_specs=[any_spec, any_spec],
        scratch_shapes=[
            pltpu.VMEM((NBUF, MAX_SEQ_LEN, HEAD_DIM), jnp.float32),
            pltpu.VMEM((NROWS, HEAD_DIM), jnp.float32),
            pltpu.VMEM((NROWS, HEAD_DIM), jnp.float32),
            pltpu.SemaphoreType.DMA((NBUF,)),
            pltpu.SemaphoreType.DMA((NBUF,)),
            pltpu.SemaphoreType.DMA,
        ],
    )

    ko, vo = pl.pallas_call(
        _update_body,
        grid_spec=grid_spec,
        out_shape=[
            jax.ShapeDtypeStruct(kc.shape, kc.dtype),
            jax.ShapeDtypeStruct(vc.shape, vc.dtype),
        ],
        compiler_params=pltpu.CompilerParams(
            vmem_limit_bytes=100 * 1024 * 1024,
        ),
    )(pos, kc, vc, kv, vv)

    return (ko.reshape(k_cache.shape), vo.reshape(v_cache.shape))
